# baseline (speedup 1.0000x reference)
import functools
import math

import jax
import jax.numpy as jnp
import numpy as np
from jax import lax
from jax.experimental import pallas as pl
from jax.experimental.pallas import tpu as pltpu
from jax.experimental.pallas import tpu_sc as plsc

D, H, Wd = 48, 48, 48
BATCH = 48
K = 27
CH = 128
GRP = 5


def _sc_gather(table, nkeys, feats_pad, n_pad, c_in):
    info = plsc.get_sparse_core_info()
    nc, ns = info.num_cores, info.num_subcores
    nw = nc * ns
    rows_per_w = n_pad // nw
    gr = GRP * CH
    groups = rows_per_w // gr

    mesh = plsc.VectorSubcoreMesh(core_axis_name="c", subcore_axis_name="s")

    @functools.partial(
        pl.kernel,
        mesh=mesh,
        out_type=jax.ShapeDtypeStruct((n_pad, K, c_in), jnp.float32),
        scratch_types=[
            pltpu.VMEM((rows_per_w,), jnp.int32),
            pltpu.VMEM((gr,), jnp.int32),
            pltpu.VMEM((gr, 1, c_in), jnp.float32),
            pltpu.SemaphoreType.DMA,
            pltpu.SemaphoreType.DMA,
        ],
    )
    def body(table_hbm, nkeys_hbm, feat_hbm, out_hbm, nkey_v, src_v, rows_v, s1, s2):
        wid = lax.axis_index("s") * nc + lax.axis_index("c")
        base = wid * rows_per_w

        def k_body(k, carry):
            pltpu.async_copy(
                nkeys_hbm.at[pl.ds(k * n_pad + base, rows_per_w)], nkey_v, s1
            ).wait()

            def g_body(g, carry2):
                g0 = g * gr
                hs = [
                    pltpu.async_copy(
                        table_hbm.at[nkey_v.at[pl.ds(g0 + c * CH, CH)]],
                        src_v.at[pl.ds(c * CH, CH)],
                        s1,
                    )
                    for c in range(GRP)
                ]
                for h in hs:
                    h.wait()
                hs = [
                    pltpu.async_copy(
                        feat_hbm.at[src_v.at[pl.ds(c * CH, CH)]],
                        rows_v.at[pl.ds(c * CH, CH)],
                        s2,
                    )
                    for c in range(GRP)
                ]
                for h in hs:
                    h.wait()
                pltpu.sync_copy(
                    rows_v, out_hbm.at[pl.ds(base + g0, gr), pl.ds(k, 1)]
                )
                return carry2

            return lax.fori_loop(0, groups, g_body, carry)

        lax.fori_loop(0, K, k_body, 0)

    return body(table, nkeys, feats_pad)


def _tc_matmul(gathered2d, w_stack, n_pad):
    kc, c_out = w_stack.shape
    bn = 512

    def mm(g_ref, w_ref, o_ref):
        o_ref[...] = jnp.dot(
            g_ref[...], w_ref[...], preferred_element_type=jnp.float32
        )

    return pl.pallas_call(
        mm,
        grid=(n_pad // bn,),
        in_specs=[
            pl.BlockSpec((bn, kc), lambda i: (i, 0)),
            pl.BlockSpec((kc, c_out), lambda i: (0, 0)),
        ],
        out_specs=pl.BlockSpec((bn, c_out), lambda i: (i, 0)),
        out_shape=jax.ShapeDtypeStruct((n_pad, c_out), jnp.float32),
    )(gathered2d, w_stack)


def kernel(features, coors, batch_size, W):
    n, c_in = features.shape
    c_out = W.shape[-1]
    m = BATCH * D * H * Wd

    coors = coors.astype(jnp.int32)
    bb, zz, yy, xx = coors[:, 0], coors[:, 1], coors[:, 2], coors[:, 3]
    key = ((bb * D + zz) * H + yy) * Wd + xx

    table = (
        jnp.full((m + 8,), n, jnp.int32)
        .at[key]
        .min(jnp.arange(n, dtype=jnp.int32))
    )

    offs = np.array(
        [(dz, dy, dx) for dz in (-1, 0, 1) for dy in (-1, 0, 1) for dx in (-1, 0, 1)],
        np.int32,
    )
    delta = jnp.asarray(offs[:, 0] * (H * Wd) + offs[:, 1] * Wd + offs[:, 2])
    dz, dy, dx = (jnp.asarray(offs[:, i])[:, None] for i in range(3))
    valid = (
        (zz[None, :] + dz >= 0) & (zz[None, :] + dz < D)
        & (yy[None, :] + dy >= 0) & (yy[None, :] + dy < H)
        & (xx[None, :] + dx >= 0) & (xx[None, :] + dx < Wd)
    )
    nk = jnp.where(valid, key[None, :] + delta[:, None], m)

    nw = 32
    n_pad = math.ceil(n / (nw * CH * GRP)) * (nw * CH * GRP)
    nkeys = jnp.full((K, n_pad), m, jnp.int32).at[:, :n].set(nk).reshape(-1)
    feats_pad = jnp.concatenate(
        [features, jnp.zeros((8, c_in), features.dtype)], axis=0
    ).reshape(n + 8, 1, c_in)

    gathered = _sc_gather(table, nkeys, feats_pad, n_pad, c_in)
    out_pad = _tc_matmul(
        gathered.reshape(n_pad, K * c_in), W.reshape(K * c_in, c_out), n_pad
    )
    return out_pad[:n]


# speedup vs baseline: 1.6835x; 1.6835x over previous
import functools
import math

import jax
import jax.numpy as jnp
import numpy as np
from jax import lax
from jax.experimental import pallas as pl
from jax.experimental.pallas import tpu as pltpu
from jax.experimental.pallas import tpu_sc as plsc

D, H, Wd = 48, 48, 48
BATCH = 48
K = 27
CH = 128
GRP = 5


def _sc_gather(grid, nkeys, n_pad, c_in):
    info = plsc.get_sparse_core_info()
    nc, ns = info.num_cores, info.num_subcores
    nw = nc * ns
    rows_per_w = n_pad // nw
    gr = GRP * CH
    groups = rows_per_w // gr

    mesh = plsc.VectorSubcoreMesh(core_axis_name="c", subcore_axis_name="s")

    @functools.partial(
        pl.kernel,
        mesh=mesh,
        out_type=jax.ShapeDtypeStruct((n_pad, K, c_in), jnp.float32),
        scratch_types=[
            pltpu.VMEM((rows_per_w,), jnp.int32),
            pltpu.VMEM((gr, 1, c_in), jnp.float32),
            pltpu.SemaphoreType.DMA,
            pltpu.SemaphoreType.DMA,
        ],
    )
    def body(grid_hbm, nkeys_hbm, out_hbm, nkey_v, rows_v, s1, s2):
        wid = lax.axis_index("s") * nc + lax.axis_index("c")
        base = wid * rows_per_w

        def k_body(k, carry):
            pltpu.async_copy(
                nkeys_hbm.at[pl.ds(k * n_pad + base, rows_per_w)], nkey_v, s1
            ).wait()

            def g_body(g, carry2):
                g0 = g * gr
                hs = [
                    pltpu.async_copy(
                        grid_hbm.at[nkey_v.at[pl.ds(g0 + c * CH, CH)]],
                        rows_v.at[pl.ds(c * CH, CH)],
                        s2,
                    )
                    for c in range(GRP)
                ]
                for h in hs:
                    h.wait()
                pltpu.sync_copy(
                    rows_v, out_hbm.at[pl.ds(base + g0, gr), pl.ds(k, 1)]
                )
                return carry2

            return lax.fori_loop(0, groups, g_body, carry)

        lax.fori_loop(0, K, k_body, 0)

    return body(grid, nkeys)


def _tc_matmul(gathered2d, w_stack, n_pad):
    kc, c_out = w_stack.shape
    bn = 512

    def mm(g_ref, w_ref, o_ref):
        o_ref[...] = jnp.dot(
            g_ref[...], w_ref[...], preferred_element_type=jnp.float32
        )

    return pl.pallas_call(
        mm,
        grid=(n_pad // bn,),
        in_specs=[
            pl.BlockSpec((bn, kc), lambda i: (i, 0)),
            pl.BlockSpec((kc, c_out), lambda i: (0, 0)),
        ],
        out_specs=pl.BlockSpec((bn, c_out), lambda i: (i, 0)),
        out_shape=jax.ShapeDtypeStruct((n_pad, c_out), jnp.float32),
    )(gathered2d, w_stack)


def kernel(features, coors, batch_size, W):
    n, c_in = features.shape
    c_out = W.shape[-1]
    m = BATCH * D * H * Wd

    coors = coors.astype(jnp.int32)
    bb, zz, yy, xx = coors[:, 0], coors[:, 1], coors[:, 2], coors[:, 3]
    key = ((bb * D + zz) * H + yy) * Wd + xx

    table = (
        jnp.full((m + 8,), n, jnp.int32)
        .at[key]
        .min(jnp.arange(n, dtype=jnp.int32))
    )
    feats_pad = jnp.concatenate(
        [features, jnp.zeros((1, c_in), features.dtype)], axis=0
    )
    grid = (
        jnp.zeros((m + 8, 1, c_in), features.dtype)
        .at[key, 0]
        .set(feats_pad[table[key]])
    )

    offs = np.array(
        [(dz, dy, dx) for dz in (-1, 0, 1) for dy in (-1, 0, 1) for dx in (-1, 0, 1)],
        np.int32,
    )
    delta = jnp.asarray(offs[:, 0] * (H * Wd) + offs[:, 1] * Wd + offs[:, 2])
    dz, dy, dx = (jnp.asarray(offs[:, i])[:, None] for i in range(3))
    valid = (
        (zz[None, :] + dz >= 0) & (zz[None, :] + dz < D)
        & (yy[None, :] + dy >= 0) & (yy[None, :] + dy < H)
        & (xx[None, :] + dx >= 0) & (xx[None, :] + dx < Wd)
    )
    nk = jnp.where(valid, key[None, :] + delta[:, None], m)

    nw = 32
    n_pad = math.ceil(n / (nw * CH * GRP)) * (nw * CH * GRP)
    nkeys = jnp.full((K, n_pad), m, jnp.int32).at[:, :n].set(nk).reshape(-1)

    gathered = _sc_gather(grid, nkeys, n_pad, c_in)
    out_pad = _tc_matmul(
        gathered.reshape(n_pad, K * c_in), W.reshape(K * c_in, c_out), n_pad
    )
    return out_pad[:n]
